# baseline (speedup 1.0000x reference)
import functools

import jax
import jax.numpy as jnp
from jax import lax
from jax.experimental import pallas as pl
from jax.experimental.pallas import tpu as pltpu
from jax.experimental.pallas import tpu_sc as plsc

VOCAB = 100000
BATCH = 16384
NEG = 20
DIM = 64
EPS = 1e-5
TEMP = 0.2
CTS_LAMDA = 0.5
UU_M = 2048

NC = 2
NS = 16
NW = NC * NS
CHUNK = 512
HEAD_W = UU_M // NW


def _sc_gather(big_t, uid, uid_tu, spos, sneg_flat, tpos, tneg_flat):
  bw = BATCH // NW
  mesh = plsc.VectorSubcoreMesh(core_axis_name="c", subcore_axis_name="s")

  @functools.partial(
      pl.kernel, mesh=mesh,
      compiler_params=pltpu.CompilerParams(use_tc_tiling_on_sc=False),
      out_type=[jax.ShapeDtypeStruct((BATCH, DIM), jnp.float32)] * 4
      + [jax.ShapeDtypeStruct((BATCH * NEG, DIM), jnp.float32)] * 2
      + [jax.ShapeDtypeStruct((UU_M, DIM), jnp.float32)] * 2,
      scratch_types=[
          pltpu.VMEM((45 * CHUNK,), jnp.int32),
          pltpu.VMEM((CHUNK, DIM), jnp.float32),
          pltpu.VMEM((CHUNK, DIM), jnp.float32),
          pltpu.VMEM((CHUNK, DIM), jnp.float32),
          pltpu.VMEM((HEAD_W, DIM), jnp.float32),
          pltpu.SemaphoreType.DMA,
          pltpu.SemaphoreType.DMA,
          pltpu.SemaphoreType.DMA,
          pltpu.SemaphoreType.DMA,
          pltpu.SemaphoreType.DMA,
      ],
  )
  def k(tab, uid_r, uidt_r, spos_r, sneg_r, tpos_r, tneg_r,
        su_o, tu_o, spi_o, tpi_o, sni_o, tni_o, suh_o, tuh_o,
        idx_all, buf_a, buf_b, buf_c, rowsh_v, sem_i, sem_g, sem_w0, sem_w1,
        sem_w2):
    wid = lax.axis_index("s") * NC + lax.axis_index("c")
    base = wid * bw
    idx_srcs = [(uid_r, base, 0), (spos_r, base, CHUNK),
                (tpos_r, base, 2 * CHUNK), (uidt_r, base, 43 * CHUNK)]
    idx_srcs += [(sneg_r, j * BATCH + base, (3 + j) * CHUNK)
                 for j in range(NEG)]
    idx_srcs += [(tneg_r, j * BATCH + base, (23 + j) * CHUNK)
                 for j in range(NEG)]
    pend = [pltpu.async_copy(src.at[pl.ds(soff, CHUNK)],
                             idx_all.at[pl.ds(doff, CHUNK)], sem_i)
            for src, soff, doff in idx_srcs]
    for c in pend:
      c.wait()
    hbase = wid * HEAD_W
    hslot = 44 * CHUNK
    pltpu.sync_copy(uid_r.at[pl.ds(hbase, HEAD_W)],
                    idx_all.at[pl.ds(hslot, HEAD_W)])
    pltpu.sync_copy(uidt_r.at[pl.ds(hbase, HEAD_W)],
                    idx_all.at[pl.ds(hslot + HEAD_W, HEAD_W)])
    for hoff, out in ((hslot, suh_o), (hslot + HEAD_W, tuh_o)):
      pltpu.async_copy(tab.at[idx_all.at[pl.ds(hoff, HEAD_W)]], rowsh_v,
                       sem_g).wait()
      pltpu.sync_copy(rowsh_v, out.at[pl.ds(hbase, HEAD_W)])
    tasks = [(0, su_o, base), (43 * CHUNK, tu_o, base),
             (CHUNK, spi_o, base), (2 * CHUNK, tpi_o, base)]
    tasks += [((3 + j) * CHUNK, sni_o, j * BATCH + base) for j in range(NEG)]
    tasks += [((23 + j) * CHUNK, tni_o, j * BATCH + base) for j in range(NEG)]
    bufs = (buf_a, buf_b, buf_c)
    wsems = (sem_w0, sem_w1, sem_w2)
    whandles = [None, None, None]
    for t, (ioff, out, ooff) in enumerate(tasks):
      b = t % 3
      if whandles[b] is not None:
        whandles[b].wait()
      pltpu.async_copy(tab.at[idx_all.at[pl.ds(ioff, CHUNK)]], bufs[b],
                       sem_g).wait()
      whandles[b] = pltpu.async_copy(bufs[b], out.at[pl.ds(ooff, CHUNK)],
                                     wsems[b])
    whandles[0].wait()
    whandles[1].wait()
    whandles[2].wait()

  return k(big_t, uid, uid_tu, spos, sneg_flat, tpos, tneg_flat)


def _proj(x):
  s = jnp.sum(x * x, axis=-1, keepdims=True)
  n = jnp.sqrt(jnp.maximum(s, EPS))
  return x / (1.0 + n) * 0.9


def _acosh(a):
  return jnp.log(a + jnp.sqrt(a * a - 1.0))


def _sigm_neg(z):
  e = jnp.exp(z)
  return e / (1.0 + e)


def _pdist_sq_terms(sq, nx, ny):
  arg = 1.0 + 2.0 * sq / jnp.maximum((1.0 - nx) * (1.0 - ny), EPS)
  return _acosh(jnp.maximum(arg, 1.0 + EPS))


def _alpha(nsq):
  return 0.9 / (1.0 + jnp.sqrt(jnp.maximum(nsq, EPS)))


R_PK = 256


def _fused_body(su_r, tu_r, spi_r, tpi_r, sni_r, tni_r, ssel_r, suh_r,
                tuh_r, w_r, b_r, out_r, outu_r):
  i = pl.program_id(0)
  lane = lax.broadcasted_iota(jnp.int32, (2, 2 * DIM), 1)
  halfr = lax.broadcasted_iota(jnp.int32, (2, 2 * DIM), 0)
  mskt = jnp.where((lane < DIM) == (halfr == 0), 1.0, 0.0)

  def red2(x):
    return lax.dot_general(mskt, x, (((1,), (1,)), ((), ())),
                           preferred_element_type=jnp.float32)

  def tile20(x):
    return jnp.concatenate([x] * NEG, axis=1)

  def one_side(u2, p2, n3):
    n3f = n3.reshape(NEG * R_PK, 2 * DIM)
    unf = (u2[None, :, :] * n3).reshape(NEG * R_PK, 2 * DIM)
    nu_t = red2(u2 * u2)
    np_t = red2(p2 * p2)
    dup_t = red2(u2 * p2)
    nn_t = red2(n3f * n3f)
    dun_t = red2(unf)
    au = _alpha(nu_t)
    ap = _alpha(np_t)
    an = _alpha(nn_t)
    nx = au * au * nu_t
    nyp = ap * ap * np_t
    sqp = jnp.maximum(nx + nyp - 2.0 * au * ap * dup_t, 0.0)
    d_pos = _pdist_sq_terms(sqp, nx, nyp)
    nxt = tile20(nx)
    aut = tile20(au)
    nyn = an * an * nn_t
    sqn = jnp.maximum(nxt + nyn - 2.0 * aut * an * dun_t, 0.0)
    d_neg = _pdist_sq_terms(sqn, nxt, nyn)
    lp = 2.0 * _sigm_neg(-d_pos) / TEMP
    ln = 2.0 * _sigm_neg(-d_neg) / TEMP
    jsum = lax.dot_general(jnp.exp(ln), ssel_r[...], (((1,), (0,)), ((), ())),
                           preferred_element_type=jnp.float32)
    lse = jnp.log(jnp.exp(lp) + jsum)
    return jnp.sum(lse - lp)

  part = (one_side(su_r[...], spi_r[...], sni_r[...])
          + one_side(tu_r[...], tpi_r[...], tni_r[...]))

  @pl.when(i == 0)
  def _():
    out_r[...] = jnp.zeros((1, 1), jnp.float32)
    outu_r[...] = jnp.zeros((1, 1), jnp.float32)

  out_r[...] += jnp.reshape(part, (1, 1))

  @pl.when(i < UU_M // R_UU)
  def _():
    outu_r[...] += jnp.reshape(_uu_part(i, suh_r, tuh_r, w_r, b_r), (1, 1))


def _losses(su, tu, spi, tpi, sni, tni, ssel, su_head, tu_head, W_s2t, b2):
  grid = (BATCH // 2) // R_PK
  r2 = pl.BlockSpec((R_PK, 2 * DIM), lambda i: (i, 0))
  r3 = pl.BlockSpec((NEG, R_PK, 2 * DIM), lambda i: (0, i, 0))
  acc_pair, acc_uu = pl.pallas_call(
      _fused_body,
      grid=(grid,),
      in_specs=[r2, r2, r2, r2, r3, r3,
                pl.BlockSpec((NEG * R_PK, R_PK), lambda i: (0, 0)),
                pl.BlockSpec((R_UU, DIM), lambda i: (i % (UU_M // R_UU), 0)),
                pl.BlockSpec((UU_M, DIM), lambda i: (0, 0)),
                pl.BlockSpec((DIM, DIM), lambda i: (0, 0)),
                pl.BlockSpec((1, DIM), lambda i: (0, 0))],
      out_specs=[pl.BlockSpec((1, 1), lambda i: (0, 0)),
                 pl.BlockSpec((1, 1), lambda i: (0, 0))],
      out_shape=[jax.ShapeDtypeStruct((1, 1), jnp.float32),
                 jax.ShapeDtypeStruct((1, 1), jnp.float32)],
  )(su, tu, spi, tpi, sni, tni, ssel, su_head, tu_head, W_s2t, b2)
  return acc_pair[0, 0] / BATCH, acc_uu[0, 0] / UU_M


R_UU = 256


def _uu_part(i, su_r, tu_r, w_r, b_r):
  pu = _proj(su_r[...])
  h = jnp.tanh(
      lax.dot_general(pu, w_r[...], (((1,), (0,)), ((), ())),
                      preferred_element_type=jnp.float32) + b_r[...])
  zs = _proj(h)
  zt = _proj(tu_r[...])
  ni = jnp.sum(zs * zs, -1)
  nj = jnp.sum(zt * zt, -1)
  cross = lax.dot_general(zs, zt, (((1,), (1,)), ((), ())),
                          preferred_element_type=jnp.float32)
  sq = jnp.maximum(ni[:, None] + nj[None, :] - 2.0 * cross, 0.0)
  d = _pdist_sq_terms(sq, ni[:, None], nj[None, :])
  logits = 2.0 * _sigm_neg(-(d * d) / TEMP)
  lse = jnp.log(jnp.sum(jnp.exp(logits), -1))
  col = lax.broadcasted_iota(jnp.int32, (R_UU, UU_M), 1)
  row = lax.broadcasted_iota(jnp.int32, (R_UU, UU_M), 0)
  diag = jnp.sum(jnp.where(col == i * R_UU + row, logits, 0.0), -1)
  return jnp.sum(lse - diag)


def kernel(source_u_table, source_i_table, target_u_table, target_i_table,
           W_s2t, b_s2t, user_ids, source_pos_i, source_neg_i, target_pos_i,
           target_neg_i):
  big_t = jnp.concatenate(
      [source_u_table, source_i_table, target_u_table, target_i_table], 0)
  su, tu, spi, tpi, sni_flat, tni_flat, su_head, tu_head = _sc_gather(
      big_t, user_ids, user_ids + 2 * VOCAB, source_pos_i + VOCAB,
      source_neg_i.T.reshape(-1) + VOCAB, target_pos_i + 3 * VOCAB,
      target_neg_i.T.reshape(-1) + 3 * VOCAB)
  pk = lambda x: x.reshape(x.shape[0] // 2, 2 * DIM)
  sni = sni_flat.reshape(NEG, BATCH // 2, 2 * DIM)
  tni = tni_flat.reshape(NEG, BATCH // 2, 2 * DIM)
  kk = jnp.arange(NEG * R_PK, dtype=jnp.int32) % R_PK
  ssel = (kk[:, None] == jnp.arange(R_PK, dtype=jnp.int32)[None, :]).astype(
      jnp.float32)
  loss_st, loss_uu = _losses(pk(su), pk(tu), pk(spi), pk(tpi), sni, tni,
                             ssel, su_head, tu_head, W_s2t,
                             b_s2t.reshape(1, DIM))
  return loss_st + CTS_LAMDA * loss_uu


# speedup vs baseline: 1.3901x; 1.3901x over previous
import functools

import jax
import jax.numpy as jnp
from jax import lax
from jax.experimental import pallas as pl
from jax.experimental.pallas import tpu as pltpu
from jax.experimental.pallas import tpu_sc as plsc

VOCAB = 100000
BATCH = 16384
NEG = 20
DIM = 64
EPS = 1e-5
TEMP = 0.2
CTS_LAMDA = 0.5
UU_M = 2048

NC = 2
NS = 16
NW = NC * NS
CHUNK = 512
HEAD_W = UU_M // NW


def _sc_gather(su_t, si_t, tu_t, ti_t, uid, spos, sneg_flat, tpos, tneg_flat):
  bw = BATCH // NW
  mesh = plsc.VectorSubcoreMesh(core_axis_name="c", subcore_axis_name="s")

  @functools.partial(
      pl.kernel, mesh=mesh,
      compiler_params=pltpu.CompilerParams(use_tc_tiling_on_sc=False),
      out_type=[jax.ShapeDtypeStruct((BATCH, DIM), jnp.float32)] * 4
      + [jax.ShapeDtypeStruct((BATCH * NEG, DIM), jnp.float32)] * 2
      + [jax.ShapeDtypeStruct((UU_M, DIM), jnp.float32)] * 2,
      scratch_types=[
          pltpu.VMEM((44 * CHUNK,), jnp.int32),
          pltpu.VMEM((CHUNK, DIM), jnp.float32),
          pltpu.VMEM((CHUNK, DIM), jnp.float32),
          pltpu.VMEM((CHUNK, DIM), jnp.float32),
          pltpu.VMEM((HEAD_W, DIM), jnp.float32),
          pltpu.SemaphoreType.DMA,
          pltpu.SemaphoreType.DMA,
          pltpu.SemaphoreType.DMA,
          pltpu.SemaphoreType.DMA,
          pltpu.SemaphoreType.DMA,
      ],
  )
  def k(su_tr, si_tr, tu_tr, ti_tr, uid_r, spos_r, sneg_r, tpos_r, tneg_r,
        su_o, tu_o, spi_o, tpi_o, sni_o, tni_o, suh_o, tuh_o,
        idx_all, buf_a, buf_b, buf_c, rowsh_v, sem_i, sem_g, sem_w0, sem_w1,
        sem_w2):
    wid = lax.axis_index("s") * NC + lax.axis_index("c")
    base = wid * bw
    idx_srcs = [(uid_r, base, 0), (spos_r, base, CHUNK),
                (tpos_r, base, 2 * CHUNK)]
    idx_srcs += [(sneg_r, j * BATCH + base, (3 + j) * CHUNK)
                 for j in range(NEG)]
    idx_srcs += [(tneg_r, j * BATCH + base, (23 + j) * CHUNK)
                 for j in range(NEG)]
    pend = [pltpu.async_copy(src.at[pl.ds(soff, CHUNK)],
                             idx_all.at[pl.ds(doff, CHUNK)], sem_i)
            for src, soff, doff in idx_srcs]
    for c in pend:
      c.wait()
    hbase = wid * HEAD_W
    hslot = 43 * CHUNK
    pltpu.sync_copy(uid_r.at[pl.ds(hbase, HEAD_W)],
                    idx_all.at[pl.ds(hslot, HEAD_W)])
    for tab, out in ((su_tr, suh_o), (tu_tr, tuh_o)):
      pltpu.async_copy(tab.at[idx_all.at[pl.ds(hslot, HEAD_W)]], rowsh_v,
                       sem_g).wait()
      pltpu.sync_copy(rowsh_v, out.at[pl.ds(hbase, HEAD_W)])
    tasks = [(0, su_tr, su_o, base), (0, tu_tr, tu_o, base),
             (CHUNK, si_tr, spi_o, base), (2 * CHUNK, ti_tr, tpi_o, base)]
    tasks += [((3 + j) * CHUNK, si_tr, sni_o, j * BATCH + base)
              for j in range(NEG)]
    tasks += [((23 + j) * CHUNK, ti_tr, tni_o, j * BATCH + base)
              for j in range(NEG)]
    bufs = (buf_a, buf_b, buf_c)
    wsems = (sem_w0, sem_w1, sem_w2)
    whandles = [None, None, None]
    for t, (ioff, tab, out, ooff) in enumerate(tasks):
      b = t % 3
      if whandles[b] is not None:
        whandles[b].wait()
      pltpu.async_copy(tab.at[idx_all.at[pl.ds(ioff, CHUNK)]], bufs[b],
                       sem_g).wait()
      whandles[b] = pltpu.async_copy(bufs[b], out.at[pl.ds(ooff, CHUNK)],
                                     wsems[b])
    whandles[0].wait()
    whandles[1].wait()
    whandles[2].wait()

  return k(su_t, si_t, tu_t, ti_t, uid, spos, sneg_flat, tpos, tneg_flat)


def _proj(x):
  s = jnp.sum(x * x, axis=-1, keepdims=True)
  n = jnp.sqrt(jnp.maximum(s, EPS))
  return x / (1.0 + n) * 0.9


def _acosh(a):
  return jnp.log(a + jnp.sqrt(a * a - 1.0))


def _sigm_neg(z):
  e = jnp.exp(z)
  return e / (1.0 + e)


def _pdist_sq_terms(sq, nx, ny):
  arg = 1.0 + 2.0 * sq / jnp.maximum((1.0 - nx) * (1.0 - ny), EPS)
  return _acosh(jnp.maximum(arg, 1.0 + EPS))


def _alpha(nsq):
  return 0.9 / (1.0 + jnp.sqrt(jnp.maximum(nsq, EPS)))


R_PK = 256


def _fused_body(su_r, tu_r, spi_r, tpi_r, sni_r, tni_r, ssel_r, suh_r,
                tuh_r, w_r, b_r, out_r, outu_r):
  i = pl.program_id(0)
  lane = lax.broadcasted_iota(jnp.int32, (2, 2 * DIM), 1)
  halfr = lax.broadcasted_iota(jnp.int32, (2, 2 * DIM), 0)
  mskt = jnp.where((lane < DIM) == (halfr == 0), 1.0, 0.0)

  def red2(x):
    return lax.dot_general(mskt, x, (((1,), (1,)), ((), ())),
                           preferred_element_type=jnp.float32)

  def tile20(x):
    return jnp.concatenate([x] * NEG, axis=1)

  def one_side(u2, p2, n3):
    n3f = n3.reshape(NEG * R_PK, 2 * DIM)
    unf = (u2[None, :, :] * n3).reshape(NEG * R_PK, 2 * DIM)
    nu_t = red2(u2 * u2)
    np_t = red2(p2 * p2)
    dup_t = red2(u2 * p2)
    nn_t = red2(n3f * n3f)
    dun_t = red2(unf)
    au = _alpha(nu_t)
    ap = _alpha(np_t)
    an = _alpha(nn_t)
    nx = au * au * nu_t
    nyp = ap * ap * np_t
    sqp = jnp.maximum(nx + nyp - 2.0 * au * ap * dup_t, 0.0)
    d_pos = _pdist_sq_terms(sqp, nx, nyp)
    nxt = tile20(nx)
    aut = tile20(au)
    nyn = an * an * nn_t
    sqn = jnp.maximum(nxt + nyn - 2.0 * aut * an * dun_t, 0.0)
    d_neg = _pdist_sq_terms(sqn, nxt, nyn)
    lp = 2.0 * _sigm_neg(-d_pos) / TEMP
    ln = 2.0 * _sigm_neg(-d_neg) / TEMP
    jsum = lax.dot_general(jnp.exp(ln), ssel_r[...], (((1,), (0,)), ((), ())),
                           preferred_element_type=jnp.float32)
    lse = jnp.log(jnp.exp(lp) + jsum)
    return jnp.sum(lse - lp)

  part = (one_side(su_r[...], spi_r[...], sni_r[...])
          + one_side(tu_r[...], tpi_r[...], tni_r[...]))

  @pl.when(i == 0)
  def _():
    out_r[...] = jnp.zeros((1, 1), jnp.float32)
    outu_r[...] = jnp.zeros((1, 1), jnp.float32)

  out_r[...] += jnp.reshape(part, (1, 1))

  @pl.when(i < UU_M // R_UU)
  def _():
    outu_r[...] += jnp.reshape(_uu_part(i, suh_r, tuh_r, w_r, b_r), (1, 1))


def _losses(su, tu, spi, tpi, sni, tni, ssel, su_head, tu_head, W_s2t, b2):
  grid = (BATCH // 2) // R_PK
  r2 = pl.BlockSpec((R_PK, 2 * DIM), lambda i: (i, 0))
  r3 = pl.BlockSpec((NEG, R_PK, 2 * DIM), lambda i: (0, i, 0))
  acc_pair, acc_uu = pl.pallas_call(
      _fused_body,
      grid=(grid,),
      in_specs=[r2, r2, r2, r2, r3, r3,
                pl.BlockSpec((NEG * R_PK, R_PK), lambda i: (0, 0)),
                pl.BlockSpec((R_UU, DIM), lambda i: (i % (UU_M // R_UU), 0)),
                pl.BlockSpec((UU_M, DIM), lambda i: (0, 0)),
                pl.BlockSpec((DIM, DIM), lambda i: (0, 0)),
                pl.BlockSpec((1, DIM), lambda i: (0, 0))],
      out_specs=[pl.BlockSpec((1, 1), lambda i: (0, 0)),
                 pl.BlockSpec((1, 1), lambda i: (0, 0))],
      out_shape=[jax.ShapeDtypeStruct((1, 1), jnp.float32),
                 jax.ShapeDtypeStruct((1, 1), jnp.float32)],
  )(su, tu, spi, tpi, sni, tni, ssel, su_head, tu_head, W_s2t, b2)
  return acc_pair[0, 0] / BATCH, acc_uu[0, 0] / UU_M


R_UU = 256


def _uu_part(i, su_r, tu_r, w_r, b_r):
  pu = _proj(su_r[...])
  h = jnp.tanh(
      lax.dot_general(pu, w_r[...], (((1,), (0,)), ((), ())),
                      preferred_element_type=jnp.float32) + b_r[...])
  zs = _proj(h)
  zt = _proj(tu_r[...])
  ni = jnp.sum(zs * zs, -1)
  nj = jnp.sum(zt * zt, -1)
  cross = lax.dot_general(zs, zt, (((1,), (1,)), ((), ())),
                          preferred_element_type=jnp.float32)
  sq = jnp.maximum(ni[:, None] + nj[None, :] - 2.0 * cross, 0.0)
  d = _pdist_sq_terms(sq, ni[:, None], nj[None, :])
  logits = 2.0 * _sigm_neg(-(d * d) / TEMP)
  lse = jnp.log(jnp.sum(jnp.exp(logits), -1))
  col = lax.broadcasted_iota(jnp.int32, (R_UU, UU_M), 1)
  row = lax.broadcasted_iota(jnp.int32, (R_UU, UU_M), 0)
  diag = jnp.sum(jnp.where(col == i * R_UU + row, logits, 0.0), -1)
  return jnp.sum(lse - diag)


def kernel(source_u_table, source_i_table, target_u_table, target_i_table,
           W_s2t, b_s2t, user_ids, source_pos_i, source_neg_i, target_pos_i,
           target_neg_i):
  su, tu, spi, tpi, sni_flat, tni_flat, su_head, tu_head = _sc_gather(
      source_u_table, source_i_table, target_u_table, target_i_table,
      user_ids, source_pos_i, source_neg_i.T.reshape(-1), target_pos_i,
      target_neg_i.T.reshape(-1))
  pk = lambda x: x.reshape(x.shape[0] // 2, 2 * DIM)
  sni = sni_flat.reshape(NEG, BATCH // 2, 2 * DIM)
  tni = tni_flat.reshape(NEG, BATCH // 2, 2 * DIM)
  kk = jnp.arange(NEG * R_PK, dtype=jnp.int32) % R_PK
  ssel = (kk[:, None] == jnp.arange(R_PK, dtype=jnp.int32)[None, :]).astype(
      jnp.float32)
  loss_st, loss_uu = _losses(pk(su), pk(tu), pk(spi), pk(tpi), sni, tni,
                             ssel, su_head, tu_head, W_s2t,
                             b_s2t.reshape(1, DIM))
  return loss_st + CTS_LAMDA * loss_uu
